# baseline (speedup 1.0000x reference)
import functools

import jax
import jax.numpy as jnp
from jax import lax
from jax.experimental import pallas as pl
from jax.experimental.pallas import tpu as pltpu
from jax.experimental.pallas import tpu_sc as plsc

D = 128
NSLOT = 11
NC, NS = 2, 16
NW = NC * NS
R = 80
BLK = 2048

CHAIN_SLOTS = [(1, 4), (4, 7), (7, 9), (9, 11)]


def _sc_gather_sum(table, idxT, b_pad):
    bpw = b_pad // NW
    nchunks = bpw // R
    dt = table.dtype
    mesh = plsc.VectorSubcoreMesh(core_axis_name="c", subcore_axis_name="s")

    @functools.partial(
        pl.kernel,
        out_type=tuple(jax.ShapeDtypeStruct((b_pad, D), dt)
                       for _ in range(5)),
        mesh=mesh,
        scratch_types=[
            pltpu.VMEM((NSLOT, bpw), jnp.int32),
            pltpu.VMEM((2, R, D), dt),
            pltpu.VMEM((8, R, D), dt),
            pltpu.SemaphoreType.DMA,
            pltpu.SemaphoreType.DMA,
            [pltpu.SemaphoreType.DMA] * 8,
        ],
        compiler_params=pltpu.CompilerParams(use_tc_tiling_on_sc=False),
    )
    def k(feat_hbm, idxT_hbm, self_hbm, ns0_hbm, ns1_hbm, ns2_hbm, ns3_hbm,
          idx_v, sbuf, nbuf, ssem, osem, csems):
        ns_hbm = [ns0_hbm, ns1_hbm, ns2_hbm, ns3_hbm]
        wid = lax.axis_index("s") * NC + lax.axis_index("c")
        base = wid * bpw
        pltpu.sync_copy(idxT_hbm.at[wid], idx_v)

        def pair(pi, carry):
            off0 = (2 * pi) * R
            selfs = [
                pltpu.async_copy(
                    feat_hbm.at[idx_v.at[0, pl.ds(off0 + h * R, R)]],
                    sbuf.at[h], ssem)
                for h in range(2)
            ]
            live = []
            for q in range(8):
                off = off0 + (q // 4) * R
                lo, hi = CHAIN_SLOTS[q % 4]
                cp = pltpu.async_copy(
                    feat_hbm.at[idx_v.at[lo, pl.ds(off, R)]], nbuf.at[q],
                    csems[q])
                live.append((q, off, lo + 1, hi, cp))
            while live:
                nxt = []
                for q, off, s, hi, cp in live:
                    cp.wait()
                    if s < hi:
                        cp2 = pltpu.async_copy(
                            feat_hbm.at[idx_v.at[s, pl.ds(off, R)]],
                            nbuf.at[q], csems[q], add=True)
                        nxt.append((q, off, s + 1, hi, cp2))
                live = nxt
            outs = []
            for h in range(2):
                selfs[h].wait()
                outs.append(pltpu.async_copy(
                    sbuf.at[h], self_hbm.at[pl.ds(base + off0 + h * R, R)],
                    osem))
            for q in range(8):
                outs.append(pltpu.async_copy(
                    nbuf.at[q],
                    ns_hbm[q % 4].at[pl.ds(base + off0 + (q // 4) * R, R)],
                    osem))
            for cp in outs:
                cp.wait()
            return carry

        lax.fori_loop(0, nchunks // 2, pair, 0)

    return k(table, idxT)


def _tc_combine(self_rows, partials, w1, w2, b):
    def body(x1, p0, p1, p2, p3, w1r, w2r, o):
        acc = jnp.dot(x1[...], w1r[...], preferred_element_type=jnp.float32)
        ns = (p0[...] + p1[...]) + (p2[...] + p3[...])
        acc = acc + jnp.dot(ns, w2r[...],
                            preferred_element_type=jnp.float32) * \
            jnp.float32(0.1)
        o[...] = jnp.maximum(acc, 0.0)

    row_spec = pl.BlockSpec((BLK, D), lambda i: (i, 0))
    w_spec = pl.BlockSpec((D, D), lambda i: (0, 0))
    return pl.pallas_call(
        body,
        grid=((b + BLK - 1) // BLK,),
        in_specs=[row_spec] * 5 + [w_spec] * 2,
        out_specs=row_spec,
        out_shape=jax.ShapeDtypeStruct((b, D), jnp.float32),
    )(self_rows, *partials, w1, w2)


def kernel(features, weight, nodes, neigh_idx):
    b = nodes.shape[0]
    step = NW * R * 2
    b_pad = ((b + step - 1) // step) * step

    idx_all = jnp.concatenate(
        [nodes[:, None].astype(jnp.int32), neigh_idx.astype(jnp.int32)],
        axis=1).T
    idxT = jnp.pad(idx_all, ((0, 0), (0, b_pad - b)))
    idxT = idxT.reshape(NSLOT, NW, b_pad // NW).transpose(1, 0, 2)

    self_rows, *partials = _sc_gather_sum(features, idxT, b_pad)
    return _tc_combine(self_rows, partials, weight[:D], weight[D:], b)


# speedup vs baseline: 2.9242x; 2.9242x over previous
import functools

import jax
import jax.numpy as jnp
from jax import lax
from jax.experimental import pallas as pl
from jax.experimental.pallas import tpu as pltpu
from jax.experimental.pallas import tpu_sc as plsc

D = 128
NSLOT = 11
NC, NS = 2, 16
NW = NC * NS
R = 112
BLK = 2048


def _sc_gather_sum(table, idxT, b_pad):
    bpw = b_pad // NW
    nchunks = bpw // R
    dt = table.dtype
    mesh = plsc.VectorSubcoreMesh(core_axis_name="c", subcore_axis_name="s")

    @functools.partial(
        pl.kernel,
        out_type=(jax.ShapeDtypeStruct((b_pad, D), dt),
                  jax.ShapeDtypeStruct((b_pad, D), dt),
                  jax.ShapeDtypeStruct((b_pad, D), dt)),
        mesh=mesh,
        scratch_types=[
            pltpu.VMEM((NSLOT, bpw), jnp.int32),
            pltpu.VMEM((2 * R, D), dt),
            pltpu.VMEM((4, R, D), dt),
            pltpu.SemaphoreType.DMA,
            pltpu.SemaphoreType.DMA,
            [pltpu.SemaphoreType.DMA] * 4,
        ],
        compiler_params=pltpu.CompilerParams(use_tc_tiling_on_sc=False),
    )
    def k(feat_hbm, idxT_hbm, self_hbm, nsa_hbm, nsb_hbm, idx_v, sbuf, nbuf,
          ssem, osem, csems):
        wid = lax.axis_index("s") * NC + lax.axis_index("c")
        base = wid * bpw
        pltpu.sync_copy(idxT_hbm.at[wid], idx_v)

        def chain_idx(q, off0, j):
            off = off0 + (q // 2) * R
            slot = 1 + (q % 2) * 5 + j
            return idx_v.at[slot, pl.ds(off, R)]

        def pair(pi, carry):
            off0 = (2 * pi) * R
            scp = pltpu.async_copy(
                feat_hbm.at[idx_v.at[0, pl.ds(off0, 2 * R)]], sbuf, ssem)
            prev = [
                pltpu.async_copy(
                    feat_hbm.at[chain_idx(q, off0, 0)], nbuf.at[q], csems[q])
                for q in range(4)
            ]
            for j in range(1, 5):
                for q in range(4):
                    prev[q].wait()
                nxt = [
                    pltpu.async_copy(
                        feat_hbm.at[chain_idx(q, off0, j)], nbuf.at[q],
                        csems[q], add=True)
                    for q in range(4)
                ]
                prev = nxt
            scp.wait()
            outs = [
                pltpu.async_copy(
                    sbuf, self_hbm.at[pl.ds(base + off0, 2 * R)], osem),
            ]
            for q in range(4):
                prev[q].wait()
                dst = nsa_hbm if q % 2 == 0 else nsb_hbm
                outs.append(pltpu.async_copy(
                    nbuf.at[q],
                    dst.at[pl.ds(base + off0 + (q // 2) * R, R)], osem))
            for cp in outs:
                cp.wait()
            return carry

        lax.fori_loop(0, nchunks // 2, pair, 0)

    return k(table, idxT)


def _tc_combine(self_rows, nsa, nsb, w1, w2, b):
    def body(x1, x2, x3, w1r, w2r, o):
        acc = jnp.dot(x1[...], w1r[...], preferred_element_type=jnp.float32)
        acc = acc + jnp.dot(x2[...] + x3[...], w2r[...],
                            preferred_element_type=jnp.float32) * \
            jnp.float32(0.1)
        o[...] = jnp.maximum(acc, 0.0)

    row_spec = pl.BlockSpec((BLK, D), lambda i: (i, 0))
    w_spec = pl.BlockSpec((D, D), lambda i: (0, 0))
    return pl.pallas_call(
        body,
        grid=((b + BLK - 1) // BLK,),
        in_specs=[row_spec] * 3 + [w_spec] * 2,
        out_specs=row_spec,
        out_shape=jax.ShapeDtypeStruct((b, D), jnp.float32),
    )(self_rows, nsa, nsb, w1, w2)


def kernel(features, weight, nodes, neigh_idx):
    b = nodes.shape[0]
    step = NW * R * 2
    b_pad = ((b + step - 1) // step) * step

    idx_all = jnp.concatenate(
        [nodes[:, None].astype(jnp.int32), neigh_idx.astype(jnp.int32)],
        axis=1).T
    idxT = jnp.pad(idx_all, ((0, 0), (0, b_pad - b)))
    idxT = idxT.reshape(NSLOT, NW, b_pad // NW).transpose(1, 0, 2)

    self_rows, nsa, nsb = _sc_gather_sum(features, idxT, b_pad)
    return _tc_combine(self_rows, nsa, nsb, weight[:D], weight[D:], b)


# speedup vs baseline: 2.9780x; 1.0184x over previous
import functools

import jax
import jax.numpy as jnp
from jax import lax
from jax.experimental import pallas as pl
from jax.experimental.pallas import tpu as pltpu
from jax.experimental.pallas import tpu_sc as plsc

D = 128
NSLOT = 11
NC, NS = 2, 16
NW = NC * NS
R = 112
BLK = 4096


def _sc_gather_sum(table, idxT, b_pad):
    bpw = b_pad // NW
    nchunks = bpw // R
    dt = table.dtype
    mesh = plsc.VectorSubcoreMesh(core_axis_name="c", subcore_axis_name="s")

    @functools.partial(
        pl.kernel,
        out_type=(jax.ShapeDtypeStruct((b_pad, D), dt),
                  jax.ShapeDtypeStruct((b_pad, D), dt),
                  jax.ShapeDtypeStruct((b_pad, D), dt)),
        mesh=mesh,
        scratch_types=[
            pltpu.VMEM((NSLOT, bpw), jnp.int32),
            pltpu.VMEM((2, R, D), dt),
            pltpu.VMEM((4, R, D), dt),
            pltpu.SemaphoreType.DMA,
            pltpu.SemaphoreType.DMA,
            [pltpu.SemaphoreType.DMA] * 4,
        ],
        compiler_params=pltpu.CompilerParams(use_tc_tiling_on_sc=False),
    )
    def k(feat_hbm, idxT_hbm, self_hbm, nsa_hbm, nsb_hbm, idx_v, sbuf, nbuf,
          ssem, osem, csems):
        wid = lax.axis_index("s") * NC + lax.axis_index("c")
        base = wid * bpw
        pltpu.sync_copy(idxT_hbm.at[wid], idx_v)

        def chain_idx(q, off0, j):
            off = off0 + (q // 2) * R
            slot = 1 + (q % 2) * 5 + j
            return idx_v.at[slot, pl.ds(off, R)]

        def pair(pi, carry):
            off0 = (2 * pi) * R
            scps = [
                pltpu.async_copy(
                    feat_hbm.at[idx_v.at[0, pl.ds(off0 + h * R, R)]],
                    sbuf.at[h], ssem)
                for h in range(2)
            ]
            prev = [
                pltpu.async_copy(
                    feat_hbm.at[chain_idx(q, off0, 0)], nbuf.at[q], csems[q])
                for q in range(4)
            ]
            for j in range(1, 5):
                for q in range(4):
                    prev[q].wait()
                nxt = [
                    pltpu.async_copy(
                        feat_hbm.at[chain_idx(q, off0, j)], nbuf.at[q],
                        csems[q], add=True)
                    for q in range(4)
                ]
                prev = nxt
            outs = []
            for h in range(2):
                scps[h].wait()
                outs.append(pltpu.async_copy(
                    sbuf.at[h], self_hbm.at[pl.ds(base + off0 + h * R, R)],
                    osem))
            for q in range(4):
                prev[q].wait()
                dst = nsa_hbm if q % 2 == 0 else nsb_hbm
                outs.append(pltpu.async_copy(
                    nbuf.at[q],
                    dst.at[pl.ds(base + off0 + (q // 2) * R, R)], osem))
            for cp in outs:
                cp.wait()
            return carry

        lax.fori_loop(0, nchunks // 2, pair, 0)

    return k(table, idxT)


def _tc_combine(self_rows, nsa, nsb, w1, w2, b):
    def body(x1, x2, x3, w1r, w2r, o):
        acc = jnp.dot(x1[...], w1r[...], preferred_element_type=jnp.float32)
        acc = acc + jnp.dot(x2[...] + x3[...], w2r[...],
                            preferred_element_type=jnp.float32) * \
            jnp.float32(0.1)
        o[...] = jnp.maximum(acc, 0.0)

    row_spec = pl.BlockSpec((BLK, D), lambda i: (i, 0))
    w_spec = pl.BlockSpec((D, D), lambda i: (0, 0))
    return pl.pallas_call(
        body,
        grid=((b + BLK - 1) // BLK,),
        in_specs=[row_spec] * 3 + [w_spec] * 2,
        out_specs=row_spec,
        out_shape=jax.ShapeDtypeStruct((b, D), jnp.float32),
    )(self_rows, nsa, nsb, w1, w2)


def kernel(features, weight, nodes, neigh_idx):
    b = nodes.shape[0]
    step = NW * R * 2
    b_pad = ((b + step - 1) // step) * step

    idx_all = jnp.concatenate(
        [nodes[:, None].astype(jnp.int32), neigh_idx.astype(jnp.int32)],
        axis=1).T
    idxT = jnp.pad(idx_all, ((0, 0), (0, b_pad - b)))
    idxT = idxT.reshape(NSLOT, NW, b_pad // NW).transpose(1, 0, 2)

    self_rows, nsa, nsb = _sc_gather_sum(features, idxT, b_pad)
    return _tc_combine(self_rows, nsa, nsb, weight[:D], weight[D:], b)


# speedup vs baseline: 3.1517x; 1.0583x over previous
import functools

import jax
import jax.numpy as jnp
from jax import lax
from jax.experimental import pallas as pl
from jax.experimental.pallas import tpu as pltpu
from jax.experimental.pallas import tpu_sc as plsc

D = 128
NSLOT = 11
NC, NS = 2, 16
NW = NC * NS
R = 112
BLK = 4096


def _sc_gather_sum(table, idxT, b_pad):
    bpw = b_pad // NW
    nchunks = bpw // R
    dt = table.dtype
    mesh = plsc.VectorSubcoreMesh(core_axis_name="c", subcore_axis_name="s")

    @functools.partial(
        pl.kernel,
        out_type=(jax.ShapeDtypeStruct((b_pad, D), dt),
                  jax.ShapeDtypeStruct((b_pad, D), dt),
                  jax.ShapeDtypeStruct((b_pad, D), dt)),
        mesh=mesh,
        scratch_types=[
            pltpu.VMEM((NSLOT, bpw), jnp.int32),
            pltpu.VMEM((2, R, D), dt),
            pltpu.VMEM((4, R, D), dt),
            pltpu.SemaphoreType.DMA,
            pltpu.SemaphoreType.DMA,
            [pltpu.SemaphoreType.DMA] * 4,
        ],
        compiler_params=pltpu.CompilerParams(use_tc_tiling_on_sc=False),
    )
    def k(feat_hbm, idxT_hbm, self_hbm, nsa_hbm, nsb_hbm, idx_v, sbuf, nbuf,
          ssem, osem, csems):
        wid = lax.axis_index("s") * NC + lax.axis_index("c")
        base = wid * bpw
        pltpu.sync_copy(idxT_hbm.at[wid], idx_v)

        def chain_idx(q, off0, j):
            off = off0 + (q // 2) * R
            slot = 1 + (q % 2) * 5 + j
            return idx_v.at[slot, pl.ds(off, R)]

        def pair(pi, carry):
            off0 = (2 * pi) * R
            scps = [
                pltpu.async_copy(
                    feat_hbm.at[idx_v.at[0, pl.ds(off0 + h * R, R)]],
                    sbuf.at[h], ssem)
                for h in range(2)
            ]
            prev = [
                pltpu.async_copy(
                    feat_hbm.at[chain_idx(q, off0, 0)], nbuf.at[q], csems[q])
                for q in range(4)
            ]
            for j in range(1, 5):
                nxt = []
                for q in range(4):
                    prev[q].wait()
                    nxt.append(pltpu.async_copy(
                        feat_hbm.at[chain_idx(q, off0, j)], nbuf.at[q],
                        csems[q], add=True))
                prev = nxt
            outs = []
            for h in range(2):
                scps[h].wait()
                outs.append(pltpu.async_copy(
                    sbuf.at[h], self_hbm.at[pl.ds(base + off0 + h * R, R)],
                    osem))
            for q in range(4):
                prev[q].wait()
                dst = nsa_hbm if q % 2 == 0 else nsb_hbm
                outs.append(pltpu.async_copy(
                    nbuf.at[q],
                    dst.at[pl.ds(base + off0 + (q // 2) * R, R)], osem))
            for cp in outs:
                cp.wait()
            return carry

        lax.fori_loop(0, nchunks // 2, pair, 0)

    return k(table, idxT)


def _tc_combine(self_rows, nsa, nsb, w1, w2, b):
    def body(x1, x2, x3, w1r, w2r, o):
        acc = jnp.dot(x1[...], w1r[...], preferred_element_type=jnp.float32)
        acc = acc + jnp.dot(x2[...] + x3[...], w2r[...],
                            preferred_element_type=jnp.float32) * \
            jnp.float32(0.1)
        o[...] = jnp.maximum(acc, 0.0)

    row_spec = pl.BlockSpec((BLK, D), lambda i: (i, 0))
    w_spec = pl.BlockSpec((D, D), lambda i: (0, 0))
    return pl.pallas_call(
        body,
        grid=((b + BLK - 1) // BLK,),
        in_specs=[row_spec] * 3 + [w_spec] * 2,
        out_specs=row_spec,
        out_shape=jax.ShapeDtypeStruct((b, D), jnp.float32),
    )(self_rows, nsa, nsb, w1, w2)


def kernel(features, weight, nodes, neigh_idx):
    b = nodes.shape[0]
    step = NW * R * 2
    b_pad = ((b + step - 1) // step) * step

    idx_all = jnp.concatenate(
        [nodes[:, None].astype(jnp.int32), neigh_idx.astype(jnp.int32)],
        axis=1).T
    idxT = jnp.pad(idx_all, ((0, 0), (0, b_pad - b)))
    idxT = idxT.reshape(NSLOT, NW, b_pad // NW).transpose(1, 0, 2)

    self_rows, nsa, nsb = _sc_gather_sum(features, idxT, b_pad)
    return _tc_combine(self_rows, nsa, nsb, weight[:D], weight[D:], b)
